# baseline (speedup 1.0000x reference)
import functools

import jax
import jax.numpy as jnp
import numpy as np
from jax import lax
from jax.experimental import pallas as pl
from jax.experimental.pallas import tpu as pltpu
from jax.experimental.pallas import tpu_sc as plsc

B = 16384
D = 128
L = 16
CHUNK = 128
KD = 16
_SQRT_MAGIC = np.int32(0x1FBD1DF5)


def _sqrt16(x):
    xi = lax.bitcast_convert_type(x, jnp.int32)
    y = lax.bitcast_convert_type(_SQRT_MAGIC + (xi >> 1), jnp.float32)
    for _ in range(3):
        y = 0.5 * (y + x / y)
    return y


def _make_kernel():
    info = plsc.get_sparse_core_info()
    nc, ns = info.num_cores, info.num_subcores
    nw = nc * ns
    b_per_w = B // nw
    n_chunks = b_per_w // CHUNK

    mesh = plsc.VectorSubcoreMesh(core_axis_name="c", subcore_axis_name="s")

    @functools.partial(
        pl.kernel,
        mesh=mesh,
        compiler_params=pltpu.CompilerParams(
            needs_layout_passes=False, disable_bounds_checks=True),
        out_type=jax.ShapeDtypeStruct((B,), jnp.float32),
        scratch_types=[
            pltpu.VMEM((b_per_w,), jnp.int32),
            pltpu.VMEM((b_per_w,), jnp.int32),
            pltpu.VMEM((b_per_w,), jnp.int32),
            pltpu.VMEM((CHUNK, D), jnp.float32),
            pltpu.VMEM((CHUNK, D), jnp.float32),
            pltpu.VMEM((CHUNK, D), jnp.float32),
            pltpu.VMEM((CHUNK, D), jnp.float32),
            pltpu.VMEM((CHUNK, D), jnp.float32),
            pltpu.VMEM((CHUNK, D), jnp.float32),
            pltpu.VMEM((b_per_w,), jnp.float32),
            pltpu.SemaphoreType.DMA,
            pltpu.SemaphoreType.DMA,
        ],
    )
    def trans_e(h_hbm, r_hbm, t_hbm, en_hbm, rel_hbm, out_hbm,
                hix, rix, tix,
                hbuf0, rbuf0, tbuf0, hbuf1, rbuf1, tbuf1,
                ssq, sem0, sem1):
        wid = lax.axis_index("s") * nc + lax.axis_index("c")
        base = wid * b_per_w
        lane = jnp.arange(L, dtype=jnp.int32)

        bufs = ((hbuf0, rbuf0, tbuf0), (hbuf1, rbuf1, tbuf1))
        sems = (sem0, sem1)

        pltpu.sync_copy(h_hbm.at[pl.ds(base, b_per_w)], hix)
        pltpu.sync_copy(r_hbm.at[pl.ds(base, b_per_w)], rix)
        pltpu.sync_copy(t_hbm.at[pl.ds(base, b_per_w)], tix)

        def fire(c):
            hbuf, rbuf, tbuf = bufs[c % 2]
            sem = sems[c % 2]
            sl = pl.ds(c * CHUNK, CHUNK)
            return (
                pltpu.async_copy(en_hbm.at[hix.at[sl]], hbuf, sem),
                pltpu.async_copy(rel_hbm.at[rix.at[sl]], rbuf, sem),
                pltpu.async_copy(en_hbm.at[tix.at[sl]], tbuf, sem),
            )

        cps = [fire(0), fire(1)]

        for c in range(n_chunks):
            for cp in cps[c % 2]:
                cp.wait()
            hbuf, rbuf, tbuf = bufs[c % 2]

            def group_body(g, _, hbuf=hbuf, rbuf=rbuf, tbuf=tbuf, c=c):
                rows = lane + g * L
                zero = jnp.zeros((L,), jnp.float32)

                @plsc.parallel_loop(0, D // KD, carry=(lane,) + (zero,) * 4)
                def carry(it, cr, rows=rows, hbuf=hbuf, rbuf=rbuf, tbuf=tbuf):
                    col0, *a = cr
                    for k in range(KD):
                        col = (col0 + k) & (D - 1)
                        tv = plsc.load_gather(tbuf, [rows, col])
                        hv = plsc.load_gather(hbuf, [rows, col])
                        rv = plsc.load_gather(rbuf, [rows, col])
                        dd = tv - hv - rv
                        a[k % 4] = a[k % 4] + dd * dd
                    return (col0 + KD, *a)

                s = tuple(carry[1:])
                while len(s) > 1:
                    s = tuple(s[i] + s[i + 1] for i in range(0, len(s), 2))
                ssq[pl.ds(c * CHUNK + g * L, L)] = _sqrt16(s[0])
                return _

            lax.fori_loop(0, CHUNK // L, group_body, None)

            if c + 2 < n_chunks:
                cps[c % 2] = fire(c + 2)

        pltpu.sync_copy(ssq, out_hbm.at[pl.ds(base, b_per_w)])

    return trans_e


_trans_e = _make_kernel()


def kernel(h_batch, r_batch, t_batch, en_embedding, rel_embedding):
    if h_batch.dtype != jnp.int32:
        h_batch = h_batch.astype(jnp.int32)
        r_batch = r_batch.astype(jnp.int32)
        t_batch = t_batch.astype(jnp.int32)
    return _trans_e(h_batch, r_batch, t_batch, en_embedding, rel_embedding)


# speedup vs baseline: 1.0088x; 1.0088x over previous
import functools

import jax
import jax.numpy as jnp
import numpy as np
from jax import lax
from jax.experimental import pallas as pl
from jax.experimental.pallas import tpu as pltpu
from jax.experimental.pallas import tpu_sc as plsc

B = 16384
D = 128
L = 16
CHUNK = 128
KD = 8
_SQRT_MAGIC = np.int32(0x1FBD1DF5)


def _sqrt16(x):
    xi = lax.bitcast_convert_type(x, jnp.int32)
    y = lax.bitcast_convert_type(_SQRT_MAGIC + (xi >> 1), jnp.float32)
    for _ in range(3):
        y = 0.5 * (y + x / y)
    return y


def _make_kernel():
    info = plsc.get_sparse_core_info()
    nc, ns = info.num_cores, info.num_subcores
    nw = nc * ns
    b_per_w = B // nw
    n_chunks = b_per_w // CHUNK

    mesh = plsc.VectorSubcoreMesh(core_axis_name="c", subcore_axis_name="s")

    @functools.partial(
        pl.kernel,
        mesh=mesh,
        compiler_params=pltpu.CompilerParams(
            needs_layout_passes=False, disable_bounds_checks=True),
        out_type=jax.ShapeDtypeStruct((B,), jnp.float32),
        scratch_types=[
            pltpu.VMEM((b_per_w,), jnp.int32),
            pltpu.VMEM((b_per_w,), jnp.int32),
            pltpu.VMEM((b_per_w,), jnp.int32),
            pltpu.VMEM((CHUNK, D), jnp.float32),
            pltpu.VMEM((CHUNK, D), jnp.float32),
            pltpu.VMEM((CHUNK, D), jnp.float32),
            pltpu.VMEM((CHUNK, D), jnp.float32),
            pltpu.VMEM((CHUNK, D), jnp.float32),
            pltpu.VMEM((CHUNK, D), jnp.float32),
            pltpu.VMEM((b_per_w,), jnp.float32),
            pltpu.SemaphoreType.DMA,
            pltpu.SemaphoreType.DMA,
        ],
    )
    def trans_e(h_hbm, r_hbm, t_hbm, en_hbm, rel_hbm, out_hbm,
                hix, rix, tix,
                hbuf0, rbuf0, tbuf0, hbuf1, rbuf1, tbuf1,
                ssq, sem0, sem1):
        wid = lax.axis_index("s") * nc + lax.axis_index("c")
        base = wid * b_per_w
        lane = jnp.arange(L, dtype=jnp.int32)

        bufs = ((hbuf0, rbuf0, tbuf0), (hbuf1, rbuf1, tbuf1))
        sems = (sem0, sem1)

        pltpu.sync_copy(h_hbm.at[pl.ds(base, b_per_w)], hix)
        pltpu.sync_copy(r_hbm.at[pl.ds(base, b_per_w)], rix)
        pltpu.sync_copy(t_hbm.at[pl.ds(base, b_per_w)], tix)

        def fire(c):
            hbuf, rbuf, tbuf = bufs[c % 2]
            sem = sems[c % 2]
            sl = pl.ds(c * CHUNK, CHUNK)
            return (
                pltpu.async_copy(en_hbm.at[hix.at[sl]], hbuf, sem),
                pltpu.async_copy(rel_hbm.at[rix.at[sl]], rbuf, sem),
                pltpu.async_copy(en_hbm.at[tix.at[sl]], tbuf, sem),
            )

        cps = [fire(0), fire(1)]

        for c in range(n_chunks):
            with jax.named_scope(f"dma_wait_{c}"):
                for cp in cps[c % 2]:
                    cp.wait()
            hbuf, rbuf, tbuf = bufs[c % 2]

            def group_body(g, _, hbuf=hbuf, rbuf=rbuf, tbuf=tbuf, c=c):
                rows = lane + g * L
                zero = jnp.zeros((L,), jnp.float32)

                @plsc.parallel_loop(0, D // KD, carry=(lane,) + (zero,) * 4)
                def carry(it, cr, rows=rows, hbuf=hbuf, rbuf=rbuf, tbuf=tbuf):
                    col0, *a = cr
                    for k in range(KD):
                        col = (col0 + k) & (D - 1)
                        tv = plsc.load_gather(tbuf, [rows, col])
                        hv = plsc.load_gather(hbuf, [rows, col])
                        rv = plsc.load_gather(rbuf, [rows, col])
                        dd = tv - hv - rv
                        a[k % 4] = a[k % 4] + dd * dd
                    return (col0 + KD, *a)

                s = tuple(carry[1:])
                while len(s) > 1:
                    s = tuple(s[i] + s[i + 1] for i in range(0, len(s), 2))
                ssq[pl.ds(c * CHUNK + g * L, L)] = _sqrt16(s[0])
                return _

            lax.fori_loop(0, CHUNK // L, group_body, None)

            if c + 2 < n_chunks:
                cps[c % 2] = fire(c + 2)

        pltpu.sync_copy(ssq, out_hbm.at[pl.ds(base, b_per_w)])

    return trans_e


_trans_e = _make_kernel()


def kernel(h_batch, r_batch, t_batch, en_embedding, rel_embedding):
    if h_batch.dtype != jnp.int32:
        h_batch = h_batch.astype(jnp.int32)
        r_batch = r_batch.astype(jnp.int32)
        t_batch = t_batch.astype(jnp.int32)
    return _trans_e(h_batch, r_batch, t_batch, en_embedding, rel_embedding)


# speedup vs baseline: 1.0649x; 1.0556x over previous
import functools

import jax
import jax.numpy as jnp
import numpy as np
from jax import lax
from jax.experimental import pallas as pl
from jax.experimental.pallas import tpu as pltpu
from jax.experimental.pallas import tpu_sc as plsc

B = 16384
D = 128
L = 16
CHUNK = 64
NBUF = 3
KD = 8
_SQRT_MAGIC = np.int32(0x1FBD1DF5)


def _sqrt16(x):
    xi = lax.bitcast_convert_type(x, jnp.int32)
    y = lax.bitcast_convert_type(_SQRT_MAGIC + (xi >> 1), jnp.float32)
    for _ in range(3):
        y = 0.5 * (y + x / y)
    return y


def _make_kernel():
    info = plsc.get_sparse_core_info()
    nc, ns = info.num_cores, info.num_subcores
    nw = nc * ns
    b_per_w = B // nw
    n_chunks = b_per_w // CHUNK

    mesh = plsc.VectorSubcoreMesh(core_axis_name="c", subcore_axis_name="s")

    @functools.partial(
        pl.kernel,
        mesh=mesh,
        compiler_params=pltpu.CompilerParams(
            needs_layout_passes=False, disable_bounds_checks=True),
        out_type=jax.ShapeDtypeStruct((B,), jnp.float32),
        scratch_types=(
            [pltpu.VMEM((b_per_w,), jnp.int32)] * 3
            + [pltpu.VMEM((CHUNK, D), jnp.float32)] * (3 * NBUF)
            + [pltpu.VMEM((b_per_w,), jnp.float32)]
            + [pltpu.SemaphoreType.DMA] * (NBUF + 1)
        ),
    )
    def trans_e(h_hbm, r_hbm, t_hbm, en_hbm, rel_hbm, out_hbm, *scratch):
        hix, rix, tix = scratch[0:3]
        bufs = tuple(tuple(scratch[3 + 3 * i:6 + 3 * i]) for i in range(NBUF))
        ssq = scratch[3 + 3 * NBUF]
        sems = scratch[4 + 3 * NBUF:4 + 4 * NBUF]
        ixsem = scratch[4 + 4 * NBUF]

        wid = lax.axis_index("s") * nc + lax.axis_index("c")
        base = wid * b_per_w
        lane = jnp.arange(L, dtype=jnp.int32)

        cpi = (pltpu.async_copy(h_hbm.at[pl.ds(base, b_per_w)], hix, ixsem),
               pltpu.async_copy(r_hbm.at[pl.ds(base, b_per_w)], rix, ixsem),
               pltpu.async_copy(t_hbm.at[pl.ds(base, b_per_w)], tix, ixsem))
        for cp in cpi:
            cp.wait()

        def fire(c):
            hbuf, rbuf, tbuf = bufs[c % NBUF]
            sem = sems[c % NBUF]
            sl = pl.ds(c * CHUNK, CHUNK)
            return (
                pltpu.async_copy(en_hbm.at[hix.at[sl]], hbuf, sem),
                pltpu.async_copy(rel_hbm.at[rix.at[sl]], rbuf, sem),
                pltpu.async_copy(en_hbm.at[tix.at[sl]], tbuf, sem),
            )

        cps = [fire(c) for c in range(NBUF)]

        for c in range(n_chunks):
            for cp in cps[c % NBUF]:
                cp.wait()
            hbuf, rbuf, tbuf = bufs[c % NBUF]

            def group_body(g, _, hbuf=hbuf, rbuf=rbuf, tbuf=tbuf, c=c):
                rows = lane + g * L
                zero = jnp.zeros((L,), jnp.float32)

                @plsc.parallel_loop(0, D // KD, carry=(lane,) + (zero,) * 4)
                def carry(it, cr, rows=rows, hbuf=hbuf, rbuf=rbuf, tbuf=tbuf):
                    col0, *a = cr
                    for k in range(KD):
                        col = (col0 + k) & (D - 1)
                        tv = plsc.load_gather(tbuf, [rows, col])
                        hv = plsc.load_gather(hbuf, [rows, col])
                        rv = plsc.load_gather(rbuf, [rows, col])
                        dd = tv - hv - rv
                        a[k % 4] = a[k % 4] + dd * dd
                    return (col0 + KD, *a)

                s = tuple(carry[1:])
                while len(s) > 1:
                    s = tuple(s[i] + s[i + 1] for i in range(0, len(s), 2))
                ssq[pl.ds(c * CHUNK + g * L, L)] = _sqrt16(s[0])
                return _

            lax.fori_loop(0, CHUNK // L, group_body, None)

            if c + NBUF < n_chunks:
                cps[c % NBUF] = fire(c + NBUF)

        pltpu.sync_copy(ssq, out_hbm.at[pl.ds(base, b_per_w)])

    return trans_e


_trans_e = _make_kernel()


def kernel(h_batch, r_batch, t_batch, en_embedding, rel_embedding):
    if h_batch.dtype != jnp.int32:
        h_batch = h_batch.astype(jnp.int32)
        r_batch = r_batch.astype(jnp.int32)
        t_batch = t_batch.astype(jnp.int32)
    return _trans_e(h_batch, r_batch, t_batch, en_embedding, rel_embedding)
